# baseline (speedup 1.0000x reference)
import functools

import jax
import jax.numpy as jnp
from jax import lax
from jax.experimental import pallas as pl
from jax.experimental.pallas import tpu as pltpu
from jax.experimental.pallas import tpu_sc as plsc

_LANES = 16
_NC = 2
_NS = 16
_NW = _NC * _NS
_CHUNK = 128
_BLK = 1000


def _ceil_to(a, m):
    return (a + m - 1) // m * m


@functools.lru_cache(maxsize=None)
def _sc_degree_kernel(ch, d, acc_rows, rpt):
    mesh = plsc.VectorSubcoreMesh(core_axis_name="c", subcore_axis_name="s")

    @functools.partial(
        pl.kernel,
        out_type=jax.ShapeDtypeStruct((_NC, acc_rows, d), jnp.float32),
        mesh=mesh,
        scratch_types=[
            pltpu.VMEM((ch, _CHUNK), jnp.int32),
            pltpu.VMEM((_CHUNK, d), jnp.float32),
            pltpu.VMEM((_CHUNK, d), jnp.float32),
            pltpu.VMEM_SHARED((acc_rows, d), jnp.float32),
        ],
    )
    def k(dst_hbm, out_hbm, dst_v, ones_v, zeros_v, acc):
        cid = lax.axis_index("c")
        sid = lax.axis_index("s")
        wid = cid * _NS + sid
        vpr = d // _LANES

        def fill(i, _):
            r = i // vpr
            c = (i % vpr) * _LANES
            zeros_v[r, pl.ds(c, _LANES)] = jnp.zeros((_LANES,), jnp.float32)
            ones_v[r, pl.ds(c, _LANES)] = jnp.ones((_LANES,), jnp.float32)
            return 0
        lax.fori_loop(0, _CHUNK * vpr, fill, 0)

        base = sid * rpt
        for q in range(rpt // _CHUNK):
            pltpu.sync_copy(zeros_v, acc.at[pl.ds(base + q * _CHUNK, _CHUNK)])
        plsc.subcore_barrier()

        for ph in range(2):
            pltpu.sync_copy(dst_hbm.at[ph, wid], dst_v)

            def body(j, _):
                pltpu.sync_copy(ones_v, acc.at[dst_v.at[j]], add=True)
                return 0
            lax.fori_loop(0, ch, body, 0)

        plsc.subcore_barrier()
        for q in range(rpt // _CHUNK):
            pltpu.sync_copy(acc.at[pl.ds(base + q * _CHUNK, _CHUNK)],
                            out_hbm.at[cid, pl.ds(base + q * _CHUNK, _CHUNK)])

    return k


def _sc_degree(dst_w, d, acc_rows, rpt):
    return _sc_degree_kernel(dst_w.shape[2], d, acc_rows, rpt)(dst_w)


@functools.lru_cache(maxsize=None)
def _sc_scatter_kernel(ch, d, acc_rows, rpt):
    mesh = plsc.VectorSubcoreMesh(core_axis_name="c", subcore_axis_name="s")

    @functools.partial(
        pl.kernel,
        out_type=jax.ShapeDtypeStruct((_NC, acc_rows, d), jnp.float32),
        mesh=mesh,
        scratch_types=[
            pltpu.VMEM((ch, _CHUNK), jnp.int32),
            pltpu.VMEM((ch, _CHUNK), jnp.int32),
            pltpu.VMEM((2, _CHUNK, d), jnp.float32),
            pltpu.VMEM_SHARED((acc_rows, d), jnp.float32),
            pltpu.SemaphoreType.DMA,
            pltpu.SemaphoreType.DMA,
        ],
    )
    def k(hs_hbm, src_hbm, dst_hbm, out_hbm, src_v, dst_v, rows_v, acc,
          sem0, sem1):
        cid = lax.axis_index("c")
        sid = lax.axis_index("s")
        wid = cid * _NS + sid
        vpr = d // _LANES

        def zbody(i, _):
            r = i // vpr
            c = (i % vpr) * _LANES
            rows_v[0, r, pl.ds(c, _LANES)] = jnp.zeros((_LANES,), jnp.float32)
            return 0
        lax.fori_loop(0, _CHUNK * vpr, zbody, 0)

        base = sid * rpt
        for q in range(rpt // _CHUNK):
            pltpu.sync_copy(rows_v.at[0],
                            acc.at[pl.ds(base + q * _CHUNK, _CHUNK)])
        plsc.subcore_barrier()

        pltpu.sync_copy(src_hbm.at[wid], src_v)
        pltpu.sync_copy(dst_hbm.at[wid], dst_v)

        pltpu.async_copy(hs_hbm.at[src_v.at[0]], rows_v.at[0], sem0)

        def body(t, _):
            j0 = 2 * t
            pltpu.async_copy(hs_hbm.at[src_v.at[j0 + 1]], rows_v.at[1], sem1)
            pltpu.make_async_copy(hs_hbm.at[src_v.at[j0]], rows_v.at[0],
                                  sem0).wait()
            pltpu.sync_copy(rows_v.at[0], acc.at[dst_v.at[j0]], add=True)

            @pl.when(t < ch // 2 - 1)
            def _():
                pltpu.async_copy(hs_hbm.at[src_v.at[j0 + 2]], rows_v.at[0],
                                 sem0)

            pltpu.make_async_copy(hs_hbm.at[src_v.at[j0 + 1]], rows_v.at[1],
                                  sem1).wait()
            pltpu.sync_copy(rows_v.at[1], acc.at[dst_v.at[j0 + 1]],
                            add=True)
            return 0
        lax.fori_loop(0, ch // 2, body, 0)

        plsc.subcore_barrier()
        for q in range(rpt // _CHUNK):
            pltpu.sync_copy(acc.at[pl.ds(base + q * _CHUNK, _CHUNK)],
                            out_hbm.at[cid, pl.ds(base + q * _CHUNK, _CHUNK)])

    return k


def _sc_scatter(hs, src_w, dst_w, acc_rows, rpt):
    return _sc_scatter_kernel(src_w.shape[1], hs.shape[1], acc_rows, rpt)(
        hs, src_w, dst_w)


def _tc_pre_body(x_ref, w_ref, c0_ref, c1_ref, hs_ref, dis_ref):
    deg = 1.0 + (c0_ref[...] + c1_ref[...])
    dis = lax.rsqrt(jnp.maximum(deg, 1.0))
    dis_ref[...] = dis
    hs_ref[...] = jnp.dot(x_ref[...], w_ref[...],
                          preferred_element_type=jnp.float32) * dis


def _tc_pre(x, w0, cols):
    n, d = x.shape
    cspec = pl.BlockSpec((_BLK, 1), lambda i: (i, 0))
    return pl.pallas_call(
        _tc_pre_body,
        grid=(n // _BLK,),
        in_specs=[
            pl.BlockSpec((_BLK, d), lambda i: (i, 0)),
            pl.BlockSpec((d, d), lambda i: (0, 0)),
            cspec, cspec,
        ],
        out_specs=[
            pl.BlockSpec((_BLK, d), lambda i: (i, 0)),
            pl.BlockSpec((_BLK, 1), lambda i: (i, 0)),
        ],
        out_shape=[
            jax.ShapeDtypeStruct((n, d), jnp.float32),
            jax.ShapeDtypeStruct((n, 1), jnp.float32),
        ],
    )(x, w0, cols[0], cols[1])


def _tc_mid_body(pa0_ref, pa1_ref, pb0_ref, pb1_ref, hs_ref, dis_ref,
                 b_ref, w_ref, out_ref):
    dis = dis_ref[...]
    agg = (pa0_ref[...] + pa1_ref[...]) + (pb0_ref[...] + pb1_ref[...])
    h = jnp.maximum(dis * (agg + hs_ref[...]) + b_ref[...], 0.0)
    out_ref[...] = jnp.dot(h, w_ref[...],
                           preferred_element_type=jnp.float32) * dis


def _tc_mid(p4, hs, dis, b, w):
    n, d = hs.shape
    bspec = pl.BlockSpec((_BLK, d), lambda i: (i, 0))
    return pl.pallas_call(
        _tc_mid_body,
        grid=(n // _BLK,),
        in_specs=[
            bspec, bspec, bspec, bspec, bspec,
            pl.BlockSpec((_BLK, 1), lambda i: (i, 0)),
            pl.BlockSpec((1, d), lambda i: (0, 0)),
            pl.BlockSpec((d, d), lambda i: (0, 0)),
        ],
        out_specs=bspec,
        out_shape=jax.ShapeDtypeStruct((n, d), jnp.float32),
    )(p4[0], p4[1], p4[2], p4[3], hs, dis, b, w)


def _tc_post_body(pa0_ref, pa1_ref, pb0_ref, pb1_ref, hs_ref, dis_ref,
                  b_ref, out_ref):
    dis = dis_ref[...]
    agg = (pa0_ref[...] + pa1_ref[...]) + (pb0_ref[...] + pb1_ref[...])
    out_ref[...] = jnp.maximum(dis * (agg + hs_ref[...]) + b_ref[...], 0.0)


def _tc_post(p4, hs, dis, b):
    n, d = hs.shape
    bspec = pl.BlockSpec((_BLK, d), lambda i: (i, 0))
    return pl.pallas_call(
        _tc_post_body,
        grid=(n // _BLK,),
        in_specs=[
            bspec, bspec, bspec, bspec, bspec,
            pl.BlockSpec((_BLK, 1), lambda i: (i, 0)),
            pl.BlockSpec((1, d), lambda i: (0, 0)),
        ],
        out_specs=bspec,
        out_shape=jax.ShapeDtypeStruct((n, d), jnp.float32),
    )(p4[0], p4[1], p4[2], p4[3], hs, dis, b)


def kernel(x, edge_index, Ws, bs):
    n, d = x.shape
    e = edge_index.shape[1]
    num_layers = Ws.shape[0]

    eh = _ceil_to(e, 2) // 2
    ch = _ceil_to(eh, _NW * _CHUNK) // (_NW * _CHUNK)
    if ch % 2:
        ch += 1
    cap = 2 * ch * _NW * _CHUNK
    pad = cap - e
    src_w = jnp.concatenate(
        [edge_index[0], jnp.zeros((pad,), edge_index.dtype)]
    ).reshape(2, _NW, ch, _CHUNK)
    dst_w = jnp.concatenate(
        [edge_index[1], jnp.full((pad,), n, edge_index.dtype)]
    ).reshape(2, _NW, ch, _CHUNK)

    rpt = _ceil_to(_ceil_to(n + 1, _NS) // _NS, _CHUNK)
    acc_rows = rpt * _NS

    pd = _sc_degree(dst_w, d, acc_rows, rpt)
    cols = (pd[0, :n, 0:1], pd[1, :n, 0:1])

    hs = dis = None
    for l in range(num_layers):
        if l == 0:
            hs, dis = _tc_pre(x, Ws[0], cols)
        else:
            hs = _tc_mid(p4, hs, dis, bs[l - 1].reshape(1, d), Ws[l])
        pa = _sc_scatter(hs, src_w[0], dst_w[0], acc_rows, rpt)
        hs_b, pa = lax.optimization_barrier((hs, pa))
        pb = _sc_scatter(hs_b, src_w[1], dst_w[1], acc_rows, rpt)
        p4 = (pa[0, :n], pa[1, :n], pb[0, :n], pb[1, :n])
    return _tc_post(p4, hs, dis, bs[num_layers - 1].reshape(1, d))


# speedup vs baseline: 2.9168x; 2.9168x over previous
import functools

import jax
import jax.numpy as jnp
from jax import lax
from jax.experimental import pallas as pl
from jax.experimental.pallas import tpu as pltpu
from jax.experimental.pallas import tpu_sc as plsc

_LANES = 16
_NC = 2
_NS = 16
_NW = _NC * _NS
_CHUNK = 128
_BLK = 1000


def _ceil_to(a, m):
    return (a + m - 1) // m * m


@functools.lru_cache(maxsize=None)
def _sc_degree_kernel(ch, d, acc_rows, rpt):
    mesh = plsc.VectorSubcoreMesh(core_axis_name="c", subcore_axis_name="s")

    @functools.partial(
        pl.kernel,
        out_type=jax.ShapeDtypeStruct((_NC, acc_rows, d), jnp.float32),
        mesh=mesh,
        scratch_types=[
            pltpu.VMEM((ch, _CHUNK), jnp.int32),
            pltpu.VMEM((_CHUNK, d), jnp.float32),
            pltpu.VMEM((_CHUNK, d), jnp.float32),
            pltpu.VMEM_SHARED((acc_rows, d), jnp.float32),
        ],
    )
    def k(dst_hbm, out_hbm, dst_v, ones_v, zeros_v, acc):
        cid = lax.axis_index("c")
        sid = lax.axis_index("s")
        wid = cid * _NS + sid
        vpr = d // _LANES

        def fill(i, _):
            r = i // vpr
            c = (i % vpr) * _LANES
            zeros_v[r, pl.ds(c, _LANES)] = jnp.zeros((_LANES,), jnp.float32)
            ones_v[r, pl.ds(c, _LANES)] = jnp.ones((_LANES,), jnp.float32)
            return 0
        lax.fori_loop(0, _CHUNK * vpr, fill, 0)

        base = sid * rpt
        for q in range(rpt // _CHUNK):
            pltpu.sync_copy(zeros_v, acc.at[pl.ds(base + q * _CHUNK, _CHUNK)])
        plsc.subcore_barrier()

        for ph in range(2):
            pltpu.sync_copy(dst_hbm.at[ph, wid], dst_v)

            def body(j, _):
                pltpu.sync_copy(ones_v, acc.at[dst_v.at[j]], add=True)
                return 0
            lax.fori_loop(0, ch, body, 0)

        plsc.subcore_barrier()
        for q in range(rpt // _CHUNK):
            pltpu.sync_copy(acc.at[pl.ds(base + q * _CHUNK, _CHUNK)],
                            out_hbm.at[cid, pl.ds(base + q * _CHUNK, _CHUNK)])

    return k


def _sc_degree(dst_w, d, acc_rows, rpt):
    return _sc_degree_kernel(dst_w.shape[2], d, acc_rows, rpt)(dst_w)


@functools.lru_cache(maxsize=None)
def _sc_scatter_kernel(ch, d, acc_rows, rpt):
    mesh = plsc.VectorSubcoreMesh(core_axis_name="c", subcore_axis_name="s")

    @functools.partial(
        pl.kernel,
        out_type=jax.ShapeDtypeStruct((_NC, acc_rows, d), jnp.float32),
        mesh=mesh,
        scratch_types=[
            pltpu.VMEM((ch, _CHUNK), jnp.int32),
            pltpu.VMEM((ch, _CHUNK), jnp.int32),
            pltpu.VMEM((2, _CHUNK, d), jnp.float32),
            pltpu.VMEM_SHARED((acc_rows, d), jnp.float32),
            pltpu.SemaphoreType.DMA,
            pltpu.SemaphoreType.DMA,
        ],
    )
    def k(hs_hbm, src_hbm, dst_hbm, out_hbm, src_v, dst_v, rows_v, acc,
          sem0, sem1):
        cid = lax.axis_index("c")
        sid = lax.axis_index("s")
        wid = cid * _NS + sid
        vpr = d // _LANES

        def zbody(i, _):
            r = i // vpr
            c = (i % vpr) * _LANES
            rows_v[0, r, pl.ds(c, _LANES)] = jnp.zeros((_LANES,), jnp.float32)
            return 0
        lax.fori_loop(0, _CHUNK * vpr, zbody, 0)

        base = sid * rpt
        for q in range(rpt // _CHUNK):
            pltpu.sync_copy(rows_v.at[0],
                            acc.at[pl.ds(base + q * _CHUNK, _CHUNK)])
        plsc.subcore_barrier()

        pltpu.sync_copy(src_hbm.at[wid], src_v)
        pltpu.sync_copy(dst_hbm.at[wid], dst_v)

        pltpu.async_copy(hs_hbm.at[src_v.at[0]], rows_v.at[0], sem0)

        def body(t, _):
            j0 = 2 * t
            pltpu.async_copy(hs_hbm.at[src_v.at[j0 + 1]], rows_v.at[1], sem1)
            pltpu.make_async_copy(hs_hbm.at[src_v.at[j0]], rows_v.at[0],
                                  sem0).wait()
            pltpu.sync_copy(rows_v.at[0], acc.at[dst_v.at[j0]], add=True)

            @pl.when(t < ch // 2 - 1)
            def _():
                pltpu.async_copy(hs_hbm.at[src_v.at[j0 + 2]], rows_v.at[0],
                                 sem0)

            pltpu.make_async_copy(hs_hbm.at[src_v.at[j0 + 1]], rows_v.at[1],
                                  sem1).wait()
            pltpu.sync_copy(rows_v.at[1], acc.at[dst_v.at[j0 + 1]],
                            add=True)
            return 0
        lax.fori_loop(0, ch // 2, body, 0)

        plsc.subcore_barrier()
        for q in range(rpt // _CHUNK):
            pltpu.sync_copy(acc.at[pl.ds(base + q * _CHUNK, _CHUNK)],
                            out_hbm.at[cid, pl.ds(base + q * _CHUNK, _CHUNK)])

    return k


def _sc_scatter(hs, src_w, dst_w, acc_rows, rpt):
    return _sc_scatter_kernel(src_w.shape[1], hs.shape[1], acc_rows, rpt)(
        hs, src_w, dst_w)


def _tc_pre_body(x_ref, w_ref, c0_ref, c1_ref, hs_ref, dis_ref):
    deg = 1.0 + (c0_ref[...] + c1_ref[...])
    dis = lax.rsqrt(jnp.maximum(deg, 1.0))
    dis_ref[...] = dis
    hs_ref[...] = jnp.dot(x_ref[...], w_ref[...],
                          preferred_element_type=jnp.float32) * dis


def _tc_pre(x, w0, cols):
    n, d = x.shape
    cspec = pl.BlockSpec((_BLK, 1), lambda i: (i, 0))
    return pl.pallas_call(
        _tc_pre_body,
        grid=(n // _BLK,),
        in_specs=[
            pl.BlockSpec((_BLK, d), lambda i: (i, 0)),
            pl.BlockSpec((d, d), lambda i: (0, 0)),
            cspec, cspec,
        ],
        out_specs=[
            pl.BlockSpec((_BLK, d), lambda i: (i, 0)),
            pl.BlockSpec((_BLK, 1), lambda i: (i, 0)),
        ],
        out_shape=[
            jax.ShapeDtypeStruct((n, d), jnp.float32),
            jax.ShapeDtypeStruct((n, 1), jnp.float32),
        ],
    )(x, w0, cols[0], cols[1])


def _tc_mid_body(pa0_ref, pa1_ref, pb0_ref, pb1_ref, hs_ref, dis_ref,
                 b_ref, w_ref, out_ref):
    dis = dis_ref[...]
    agg = (pa0_ref[...] + pa1_ref[...]) + (pb0_ref[...] + pb1_ref[...])
    h = jnp.maximum(dis * (agg + hs_ref[...]) + b_ref[...], 0.0)
    out_ref[...] = jnp.dot(h, w_ref[...],
                           preferred_element_type=jnp.float32) * dis


def _tc_mid(p4, hs, dis, b, w):
    n, d = hs.shape
    bspec = pl.BlockSpec((_BLK, d), lambda i: (i, 0))
    return pl.pallas_call(
        _tc_mid_body,
        grid=(n // _BLK,),
        in_specs=[
            bspec, bspec, bspec, bspec, bspec,
            pl.BlockSpec((_BLK, 1), lambda i: (i, 0)),
            pl.BlockSpec((1, d), lambda i: (0, 0)),
            pl.BlockSpec((d, d), lambda i: (0, 0)),
        ],
        out_specs=bspec,
        out_shape=jax.ShapeDtypeStruct((n, d), jnp.float32),
    )(p4[0], p4[1], p4[2], p4[3], hs, dis, b, w)


def _tc_post_body(pa0_ref, pa1_ref, pb0_ref, pb1_ref, hs_ref, dis_ref,
                  b_ref, out_ref):
    dis = dis_ref[...]
    agg = (pa0_ref[...] + pa1_ref[...]) + (pb0_ref[...] + pb1_ref[...])
    out_ref[...] = jnp.maximum(dis * (agg + hs_ref[...]) + b_ref[...], 0.0)


def _tc_post(p4, hs, dis, b):
    n, d = hs.shape
    bspec = pl.BlockSpec((_BLK, d), lambda i: (i, 0))
    return pl.pallas_call(
        _tc_post_body,
        grid=(n // _BLK,),
        in_specs=[
            bspec, bspec, bspec, bspec, bspec,
            pl.BlockSpec((_BLK, 1), lambda i: (i, 0)),
            pl.BlockSpec((1, d), lambda i: (0, 0)),
        ],
        out_specs=bspec,
        out_shape=jax.ShapeDtypeStruct((n, d), jnp.float32),
    )(p4[0], p4[1], p4[2], p4[3], hs, dis, b)


def kernel(x, edge_index, Ws, bs):
    n, d = x.shape
    e = edge_index.shape[1]
    num_layers = Ws.shape[0]

    eh = _ceil_to(e, 2) // 2
    ch = _ceil_to(eh, _NW * _CHUNK) // (_NW * _CHUNK)
    if ch % 2:
        ch += 1
    cap = 2 * ch * _NW * _CHUNK
    pad = cap - e

    rpt = _ceil_to(_ceil_to(n + 1, _NS) // _NS, _CHUNK)
    acc_rows = rpt * _NS

    pad_ids = jnp.arange(pad, dtype=edge_index.dtype)
    src_w = jnp.concatenate(
        [edge_index[0], pad_ids % n]
    ).reshape(2, _NW, ch, _CHUNK)
    dst_w = jnp.concatenate(
        [edge_index[1], n + pad_ids % (acc_rows - n)]
    ).reshape(2, _NW, ch, _CHUNK)

    pd = _sc_degree(dst_w, d, acc_rows, rpt)
    cols = (pd[0, :n, 0:1], pd[1, :n, 0:1])

    hs = dis = None
    for l in range(num_layers):
        if l == 0:
            hs, dis = _tc_pre(x, Ws[0], cols)
        else:
            hs = _tc_mid(p4, hs, dis, bs[l - 1].reshape(1, d), Ws[l])
        pa = _sc_scatter(hs, src_w[0], dst_w[0], acc_rows, rpt)
        hs_b, pa = lax.optimization_barrier((hs, pa))
        pb = _sc_scatter(hs_b, src_w[1], dst_w[1], acc_rows, rpt)
        p4 = (pa[0, :n], pa[1, :n], pb[0, :n], pb[1, :n])
    return _tc_post(p4, hs, dis, bs[num_layers - 1].reshape(1, d))


# speedup vs baseline: 3.0838x; 1.0573x over previous
import functools

import jax
import jax.numpy as jnp
from jax import lax
from jax.experimental import pallas as pl
from jax.experimental.pallas import tpu as pltpu
from jax.experimental.pallas import tpu_sc as plsc

_LANES = 16
_NC = 2
_NS = 16
_NW = _NC * _NS
_CHUNK = 128
_BLK = 1000


def _ceil_to(a, m):
    return (a + m - 1) // m * m


@functools.lru_cache(maxsize=None)
def _sc_degree_kernel(ch, d, acc_rows, rpt):
    mesh = plsc.VectorSubcoreMesh(core_axis_name="c", subcore_axis_name="s")

    @functools.partial(
        pl.kernel,
        out_type=jax.ShapeDtypeStruct((_NC, acc_rows, d), jnp.float32),
        mesh=mesh,
        scratch_types=[
            pltpu.VMEM((ch, _CHUNK), jnp.int32),
            pltpu.VMEM((_CHUNK, d), jnp.float32),
            pltpu.VMEM((_CHUNK, d), jnp.float32),
            pltpu.VMEM_SHARED((acc_rows, d), jnp.float32),
        ],
    )
    def k(dst_hbm, out_hbm, dst_v, ones_v, zeros_v, acc):
        cid = lax.axis_index("c")
        sid = lax.axis_index("s")
        wid = cid * _NS + sid
        vpr = d // _LANES

        def fill(i, _):
            r = i // vpr
            c = (i % vpr) * _LANES
            zeros_v[r, pl.ds(c, _LANES)] = jnp.zeros((_LANES,), jnp.float32)
            ones_v[r, pl.ds(c, _LANES)] = jnp.ones((_LANES,), jnp.float32)
            return 0
        lax.fori_loop(0, _CHUNK * vpr, fill, 0)

        base = sid * rpt
        for q in range(rpt // _CHUNK):
            pltpu.sync_copy(zeros_v, acc.at[pl.ds(base + q * _CHUNK, _CHUNK)])
        plsc.subcore_barrier()

        for ph in range(2):
            pltpu.sync_copy(dst_hbm.at[ph, wid], dst_v)

            def body(j, _):
                pltpu.sync_copy(ones_v, acc.at[dst_v.at[j]], add=True)
                return 0
            lax.fori_loop(0, ch, body, 0)

        plsc.subcore_barrier()
        for q in range(rpt // _CHUNK):
            pltpu.sync_copy(acc.at[pl.ds(base + q * _CHUNK, _CHUNK)],
                            out_hbm.at[cid, pl.ds(base + q * _CHUNK, _CHUNK)])

    return k


def _sc_degree(dst_w, d, acc_rows, rpt):
    return _sc_degree_kernel(dst_w.shape[2], d, acc_rows, rpt)(dst_w)


@functools.lru_cache(maxsize=None)
def _sc_scatter_kernel(ch, d, acc_rows, rpt):
    mesh = plsc.VectorSubcoreMesh(core_axis_name="c", subcore_axis_name="s")

    @functools.partial(
        pl.kernel,
        out_type=jax.ShapeDtypeStruct((_NC, acc_rows, d), jnp.float32),
        mesh=mesh,
        scratch_types=[
            pltpu.VMEM((ch, _CHUNK), jnp.int32),
            pltpu.VMEM((ch, _CHUNK), jnp.int32),
            pltpu.VMEM((2, _CHUNK, d), jnp.float32),
            pltpu.VMEM_SHARED((acc_rows, d), jnp.float32),
            pltpu.SemaphoreType.DMA,
            pltpu.SemaphoreType.DMA,
        ],
    )
    def k(hs_hbm, src_hbm, dst_hbm, out_hbm, src_v, dst_v, rows_v, acc,
          sem0, sem1):
        cid = lax.axis_index("c")
        sid = lax.axis_index("s")
        wid = cid * _NS + sid
        vpr = d // _LANES

        def zbody(i, _):
            r = i // vpr
            c = (i % vpr) * _LANES
            rows_v[0, r, pl.ds(c, _LANES)] = jnp.zeros((_LANES,), jnp.float32)
            return 0
        lax.fori_loop(0, _CHUNK * vpr, zbody, 0)

        base = sid * rpt
        for q in range(rpt // _CHUNK):
            pltpu.sync_copy(rows_v.at[0],
                            acc.at[pl.ds(base + q * _CHUNK, _CHUNK)])
        plsc.subcore_barrier()

        pltpu.sync_copy(src_hbm.at[wid], src_v)
        pltpu.sync_copy(dst_hbm.at[wid], dst_v)

        pltpu.async_copy(hs_hbm.at[src_v.at[0]], rows_v.at[0], sem0)

        def body(t, _):
            j0 = 2 * t
            pltpu.async_copy(hs_hbm.at[src_v.at[j0 + 1]], rows_v.at[1], sem1)
            pltpu.make_async_copy(hs_hbm.at[src_v.at[j0]], rows_v.at[0],
                                  sem0).wait()
            pltpu.sync_copy(rows_v.at[0], acc.at[dst_v.at[j0]], add=True)

            @pl.when(t < ch // 2 - 1)
            def _():
                pltpu.async_copy(hs_hbm.at[src_v.at[j0 + 2]], rows_v.at[0],
                                 sem0)

            pltpu.make_async_copy(hs_hbm.at[src_v.at[j0 + 1]], rows_v.at[1],
                                  sem1).wait()
            pltpu.sync_copy(rows_v.at[1], acc.at[dst_v.at[j0 + 1]],
                            add=True)
            return 0
        lax.fori_loop(0, ch // 2, body, 0)

        plsc.subcore_barrier()
        for q in range(rpt // _CHUNK):
            pltpu.sync_copy(acc.at[pl.ds(base + q * _CHUNK, _CHUNK)],
                            out_hbm.at[cid, pl.ds(base + q * _CHUNK, _CHUNK)])

    return k


def _sc_scatter(hs, src_w, dst_w, acc_rows, rpt):
    return _sc_scatter_kernel(src_w.shape[1], hs.shape[1], acc_rows, rpt)(
        hs, src_w, dst_w)


def _pspec0(d):
    return pl.BlockSpec((1, _BLK, d), lambda i: (0, i, 0))


def _pspec1(d):
    return pl.BlockSpec((1, _BLK, d), lambda i: (1, i, 0))


def _tc_pre_body(x_ref, w_ref, c0_ref, c1_ref, hs_ref, dis_ref):
    deg = 1.0 + (c0_ref[0, :, 0:1] + c1_ref[0, :, 0:1])
    dis = lax.rsqrt(jnp.maximum(deg, 1.0))
    dis_ref[...] = dis
    hs_ref[...] = jnp.dot(x_ref[...], w_ref[...],
                          preferred_element_type=jnp.float32) * dis


def _tc_pre(x, w0, pd):
    n, d = x.shape
    return pl.pallas_call(
        _tc_pre_body,
        grid=(n // _BLK,),
        in_specs=[
            pl.BlockSpec((_BLK, d), lambda i: (i, 0)),
            pl.BlockSpec((d, d), lambda i: (0, 0)),
            _pspec0(d), _pspec1(d),
        ],
        out_specs=[
            pl.BlockSpec((_BLK, d), lambda i: (i, 0)),
            pl.BlockSpec((_BLK, 1), lambda i: (i, 0)),
        ],
        out_shape=[
            jax.ShapeDtypeStruct((n, d), jnp.float32),
            jax.ShapeDtypeStruct((n, 1), jnp.float32),
        ],
    )(x, w0, pd, pd)


def _tc_mid_body(pa0_ref, pa1_ref, pb0_ref, pb1_ref, hs_ref, dis_ref,
                 b_ref, w_ref, out_ref):
    dis = dis_ref[...]
    agg = (pa0_ref[0] + pa1_ref[0]) + (pb0_ref[0] + pb1_ref[0])
    h = jnp.maximum(dis * (agg + hs_ref[...]) + b_ref[...], 0.0)
    out_ref[...] = jnp.dot(h, w_ref[...],
                           preferred_element_type=jnp.float32) * dis


def _tc_mid(pa, pb, hs, dis, b, w):
    n, d = hs.shape
    bspec = pl.BlockSpec((_BLK, d), lambda i: (i, 0))
    return pl.pallas_call(
        _tc_mid_body,
        grid=(n // _BLK,),
        in_specs=[
            _pspec0(d), _pspec1(d), _pspec0(d), _pspec1(d), bspec,
            pl.BlockSpec((_BLK, 1), lambda i: (i, 0)),
            pl.BlockSpec((1, d), lambda i: (0, 0)),
            pl.BlockSpec((d, d), lambda i: (0, 0)),
        ],
        out_specs=bspec,
        out_shape=jax.ShapeDtypeStruct((n, d), jnp.float32),
    )(pa, pa, pb, pb, hs, dis, b, w)


def _tc_post_body(pa0_ref, pa1_ref, pb0_ref, pb1_ref, hs_ref, dis_ref,
                  b_ref, out_ref):
    dis = dis_ref[...]
    agg = (pa0_ref[0] + pa1_ref[0]) + (pb0_ref[0] + pb1_ref[0])
    out_ref[...] = jnp.maximum(dis * (agg + hs_ref[...]) + b_ref[...], 0.0)


def _tc_post(pa, pb, hs, dis, b):
    n, d = hs.shape
    bspec = pl.BlockSpec((_BLK, d), lambda i: (i, 0))
    return pl.pallas_call(
        _tc_post_body,
        grid=(n // _BLK,),
        in_specs=[
            _pspec0(d), _pspec1(d), _pspec0(d), _pspec1(d), bspec,
            pl.BlockSpec((_BLK, 1), lambda i: (i, 0)),
            pl.BlockSpec((1, d), lambda i: (0, 0)),
        ],
        out_specs=bspec,
        out_shape=jax.ShapeDtypeStruct((n, d), jnp.float32),
    )(pa, pa, pb, pb, hs, dis, b)


def kernel(x, edge_index, Ws, bs):
    n, d = x.shape
    e = edge_index.shape[1]
    num_layers = Ws.shape[0]

    eh = _ceil_to(e, 2) // 2
    ch = _ceil_to(eh, _NW * _CHUNK) // (_NW * _CHUNK)
    if ch % 2:
        ch += 1
    cap = 2 * ch * _NW * _CHUNK
    pad = cap - e

    rpt = _ceil_to(_ceil_to(n + 1, _NS) // _NS, _CHUNK)
    acc_rows = rpt * _NS

    pad_ids = jnp.arange(pad, dtype=edge_index.dtype)
    src_w = jnp.concatenate(
        [edge_index[0], pad_ids % n]
    ).reshape(2, _NW, ch, _CHUNK)
    dst_w = jnp.concatenate(
        [edge_index[1], n + pad_ids % (acc_rows - n)]
    ).reshape(2, _NW, ch, _CHUNK)

    pd = _sc_degree(dst_w, d, acc_rows, rpt)

    hs = dis = None
    for l in range(num_layers):
        if l == 0:
            hs, dis = _tc_pre(x, Ws[0], pd)
        else:
            hs = _tc_mid(pa, pb, hs, dis, bs[l - 1].reshape(1, d), Ws[l])
        pa = _sc_scatter(hs, src_w[0], dst_w[0], acc_rows, rpt)
        pb = _sc_scatter(hs, src_w[1], dst_w[1], acc_rows, rpt)
    return _tc_post(pa, pb, hs, dis, bs[num_layers - 1].reshape(1, d))


# speedup vs baseline: 3.3268x; 1.0788x over previous
import functools

import jax
import jax.numpy as jnp
from jax import lax
from jax.experimental import pallas as pl
from jax.experimental.pallas import tpu as pltpu
from jax.experimental.pallas import tpu_sc as plsc

_LANES = 16
_NC = 2
_NS = 16
_NW = _NC * _NS
_CHUNK = 128
_BLK = 1000


def _ceil_to(a, m):
    return (a + m - 1) // m * m


@functools.lru_cache(maxsize=None)
def _sc_degree_kernel(ch, d, acc_rows, rpt):
    mesh = plsc.VectorSubcoreMesh(core_axis_name="c", subcore_axis_name="s")

    @functools.partial(
        pl.kernel,
        out_type=jax.ShapeDtypeStruct((_NC, acc_rows, d), jnp.float32),
        mesh=mesh,
        scratch_types=[
            pltpu.VMEM((ch, _CHUNK), jnp.int32),
            pltpu.VMEM((_CHUNK, d), jnp.float32),
            pltpu.VMEM((_CHUNK, d), jnp.float32),
            pltpu.VMEM_SHARED((acc_rows, d), jnp.float32),
        ],
    )
    def k(dst_hbm, out_hbm, dst_v, ones_v, zeros_v, acc):
        cid = lax.axis_index("c")
        sid = lax.axis_index("s")
        wid = cid * _NS + sid
        vpr = d // _LANES

        def fill(i, _):
            r = i // vpr
            c = (i % vpr) * _LANES
            zeros_v[r, pl.ds(c, _LANES)] = jnp.zeros((_LANES,), jnp.float32)
            ones_v[r, pl.ds(c, _LANES)] = jnp.ones((_LANES,), jnp.float32)
            return 0
        lax.fori_loop(0, _CHUNK * vpr, fill, 0)

        base = sid * rpt
        for q in range(rpt // _CHUNK):
            pltpu.sync_copy(zeros_v, acc.at[pl.ds(base + q * _CHUNK, _CHUNK)])
        plsc.subcore_barrier()

        for ph in range(2):
            pltpu.sync_copy(dst_hbm.at[ph, wid], dst_v)

            def body(j, _):
                pltpu.sync_copy(ones_v, acc.at[dst_v.at[j]], add=True)
                return 0
            lax.fori_loop(0, ch, body, 0)

        plsc.subcore_barrier()
        for q in range(rpt // _CHUNK):
            pltpu.sync_copy(acc.at[pl.ds(base + q * _CHUNK, _CHUNK)],
                            out_hbm.at[cid, pl.ds(base + q * _CHUNK, _CHUNK)])

    return k


def _sc_degree(dst_w, d, acc_rows, rpt):
    return _sc_degree_kernel(dst_w.shape[2], d, acc_rows, rpt)(dst_w)


@functools.lru_cache(maxsize=None)
def _sc_scatter_kernel(ch, d, acc_rows, rpt):
    mesh = plsc.VectorSubcoreMesh(core_axis_name="c", subcore_axis_name="s")

    @functools.partial(
        pl.kernel,
        out_type=jax.ShapeDtypeStruct((_NC, acc_rows, d), jnp.float32),
        mesh=mesh,
        scratch_types=[
            pltpu.VMEM((ch, _CHUNK), jnp.int32),
            pltpu.VMEM((ch, _CHUNK), jnp.int32),
            pltpu.VMEM((2, _CHUNK, d), jnp.float32),
            pltpu.VMEM_SHARED((acc_rows, d), jnp.float32),
            pltpu.SemaphoreType.DMA,
            pltpu.SemaphoreType.DMA,
        ],
    )
    def k(hs_hbm, src_hbm, dst_hbm, out_hbm, src_v, dst_v, rows_v, acc,
          sem0, sem1):
        cid = lax.axis_index("c")
        sid = lax.axis_index("s")
        wid = cid * _NS + sid
        vpr = d // _LANES

        def zbody(i, _):
            r = i // vpr
            c = (i % vpr) * _LANES
            rows_v[0, r, pl.ds(c, _LANES)] = jnp.zeros((_LANES,), jnp.float32)
            return 0
        lax.fori_loop(0, _CHUNK * vpr, zbody, 0)

        base = sid * rpt
        for q in range(rpt // _CHUNK):
            pltpu.sync_copy(rows_v.at[0],
                            acc.at[pl.ds(base + q * _CHUNK, _CHUNK)])
        plsc.subcore_barrier()

        pltpu.sync_copy(src_hbm.at[wid], src_v)
        pltpu.sync_copy(dst_hbm.at[wid], dst_v)

        pltpu.async_copy(hs_hbm.at[src_v.at[0]], rows_v.at[0], sem0)

        def body(t, _):
            j0 = 2 * t
            pltpu.async_copy(hs_hbm.at[src_v.at[j0 + 1]], rows_v.at[1], sem1)
            pltpu.make_async_copy(hs_hbm.at[src_v.at[j0]], rows_v.at[0],
                                  sem0).wait()
            pltpu.sync_copy(rows_v.at[0], acc.at[dst_v.at[j0]], add=True)

            @pl.when(t < ch // 2 - 1)
            def _():
                pltpu.async_copy(hs_hbm.at[src_v.at[j0 + 2]], rows_v.at[0],
                                 sem0)

            pltpu.make_async_copy(hs_hbm.at[src_v.at[j0 + 1]], rows_v.at[1],
                                  sem1).wait()
            pltpu.sync_copy(rows_v.at[1], acc.at[dst_v.at[j0 + 1]],
                            add=True)
            return 0
        lax.fori_loop(0, ch // 2, body, 0)

        plsc.subcore_barrier()
        for q in range(rpt // _CHUNK):
            pltpu.sync_copy(acc.at[pl.ds(base + q * _CHUNK, _CHUNK)],
                            out_hbm.at[cid, pl.ds(base + q * _CHUNK, _CHUNK)])

    return k


def _sc_scatter(hs, src_w, dst_w, acc_rows, rpt):
    return _sc_scatter_kernel(src_w.shape[1], hs.shape[1], acc_rows, rpt)(
        hs, src_w, dst_w)


def _pspec0(d):
    return pl.BlockSpec((1, _BLK, d), lambda i: (0, i, 0))


def _pspec1(d):
    return pl.BlockSpec((1, _BLK, d), lambda i: (1, i, 0))


def _tc_pre_body(x_ref, w_ref, c0_ref, c1_ref, hs_ref, dis_ref):
    deg = 1.0 + (c0_ref[0, :, 0:1] + c1_ref[0, :, 0:1])
    dis = lax.rsqrt(jnp.maximum(deg, 1.0))
    dis_ref[...] = dis
    hs_ref[...] = jnp.dot(x_ref[...], w_ref[...],
                          preferred_element_type=jnp.float32) * dis


def _tc_pre(x, w0, pd):
    n, d = x.shape
    return pl.pallas_call(
        _tc_pre_body,
        grid=(n // _BLK,),
        in_specs=[
            pl.BlockSpec((_BLK, d), lambda i: (i, 0)),
            pl.BlockSpec((d, d), lambda i: (0, 0)),
            _pspec0(pd.shape[2]), _pspec1(pd.shape[2]),
        ],
        out_specs=[
            pl.BlockSpec((_BLK, d), lambda i: (i, 0)),
            pl.BlockSpec((_BLK, 1), lambda i: (i, 0)),
        ],
        out_shape=[
            jax.ShapeDtypeStruct((n, d), jnp.float32),
            jax.ShapeDtypeStruct((n, 1), jnp.float32),
        ],
    )(x, w0, pd, pd)


def _tc_mid_body(pa0_ref, pa1_ref, pb0_ref, pb1_ref, hs_ref, dis_ref,
                 b_ref, w_ref, out_ref):
    dis = dis_ref[...]
    agg = (pa0_ref[0] + pa1_ref[0]) + (pb0_ref[0] + pb1_ref[0])
    h = jnp.maximum(dis * (agg + hs_ref[...]) + b_ref[...], 0.0)
    out_ref[...] = jnp.dot(h, w_ref[...],
                           preferred_element_type=jnp.float32) * dis


def _tc_mid(pa, pb, hs, dis, b, w):
    n, d = hs.shape
    bspec = pl.BlockSpec((_BLK, d), lambda i: (i, 0))
    return pl.pallas_call(
        _tc_mid_body,
        grid=(n // _BLK,),
        in_specs=[
            _pspec0(d), _pspec1(d), _pspec0(d), _pspec1(d), bspec,
            pl.BlockSpec((_BLK, 1), lambda i: (i, 0)),
            pl.BlockSpec((1, d), lambda i: (0, 0)),
            pl.BlockSpec((d, d), lambda i: (0, 0)),
        ],
        out_specs=bspec,
        out_shape=jax.ShapeDtypeStruct((n, d), jnp.float32),
    )(pa, pa, pb, pb, hs, dis, b, w)


def _tc_post_body(pa0_ref, pa1_ref, pb0_ref, pb1_ref, hs_ref, dis_ref,
                  b_ref, out_ref):
    dis = dis_ref[...]
    agg = (pa0_ref[0] + pa1_ref[0]) + (pb0_ref[0] + pb1_ref[0])
    out_ref[...] = jnp.maximum(dis * (agg + hs_ref[...]) + b_ref[...], 0.0)


def _tc_post(pa, pb, hs, dis, b):
    n, d = hs.shape
    bspec = pl.BlockSpec((_BLK, d), lambda i: (i, 0))
    return pl.pallas_call(
        _tc_post_body,
        grid=(n // _BLK,),
        in_specs=[
            _pspec0(d), _pspec1(d), _pspec0(d), _pspec1(d), bspec,
            pl.BlockSpec((_BLK, 1), lambda i: (i, 0)),
            pl.BlockSpec((1, d), lambda i: (0, 0)),
        ],
        out_specs=bspec,
        out_shape=jax.ShapeDtypeStruct((n, d), jnp.float32),
    )(pa, pa, pb, pb, hs, dis, b)


def kernel(x, edge_index, Ws, bs):
    n, d = x.shape
    e = edge_index.shape[1]
    num_layers = Ws.shape[0]

    eh = _ceil_to(e, 2) // 2
    ch = _ceil_to(eh, _NW * _CHUNK) // (_NW * _CHUNK)
    if ch % 2:
        ch += 1
    cap = 2 * ch * _NW * _CHUNK
    pad = cap - e

    rpt = _ceil_to(_ceil_to(n + 1, _NS) // _NS, _CHUNK)
    acc_rows = rpt * _NS

    pad_ids = jnp.arange(pad, dtype=edge_index.dtype)
    src_w = jnp.concatenate(
        [edge_index[0], pad_ids % n]
    ).reshape(2, _NW, ch, _CHUNK)
    dst_w = jnp.concatenate(
        [edge_index[1], n + pad_ids % (acc_rows - n)]
    ).reshape(2, _NW, ch, _CHUNK)

    pd = _sc_degree(dst_w, 64, acc_rows, rpt)

    hs = dis = None
    for l in range(num_layers):
        if l == 0:
            hs, dis = _tc_pre(x, Ws[0], pd)
        else:
            hs = _tc_mid(pa, pb, hs, dis, bs[l - 1].reshape(1, d), Ws[l])
        pa = _sc_scatter(hs, src_w[0], dst_w[0], acc_rows, rpt)
        pb = _sc_scatter(hs, src_w[1], dst_w[1], acc_rows, rpt)
    return _tc_post(pa, pb, hs, dis, bs[num_layers - 1].reshape(1, d))
